# Initial kernel scaffold; baseline (speedup 1.0000x reference)
import jax
import jax.numpy as jnp
from jax.experimental import pallas as pl


def kernel(z, edge_index, batch, z_table, w1_0, b1_0, w2_0, b2_0, g_0, bt_0, w1_1, b1_1, w2_1, b2_1, g_1, bt_1, w1_2, b1_2, w2_2, b2_2, g_2, bt_2, lin1_w, lin1_b, lin2_w, lin2_b):
    raise NotImplementedError("write your pallas kernel here")


# speedup vs baseline: 4.4581x; 4.4581x over previous
import functools

import jax
import jax.numpy as jnp
from jax import lax
from jax.experimental import pallas as pl
from jax.experimental.pallas import tpu as pltpu
from jax.experimental.pallas import tpu_sc as plsc

N = 10000
E = 320000
H = 128
NG = 64

NC = 2
NS = 16
LANES = 16


_GK = 80
_GCHUNKS = N // _GK


@functools.cache
def _sc_mesh():
    return plsc.VectorSubcoreMesh(core_axis_name="c", subcore_axis_name="s",
                                  num_cores=NC, num_subcores=NS)


@functools.cache
def _gather_rows_k():
    @functools.partial(
        pl.kernel,
        out_type=jax.ShapeDtypeStruct((N, H), jnp.float32),
        mesh=_sc_mesh(),
        scratch_types=[
            pltpu.VMEM((_GK,), jnp.int32),
            pltpu.VMEM((_GK, H), jnp.float32),
            pltpu.SemaphoreType.DMA,
        ],
    )
    def gather_k(table_hbm, z_hbm, out_hbm, idx_v, rows_v, sem):
        cid = lax.axis_index("c")
        sid = lax.axis_index("s")
        wid = sid * NC + cid

        def body(j, carry):
            c = wid + 32 * j

            @pl.when(c < _GCHUNKS)
            def _():
                off = c * _GK
                pltpu.sync_copy(z_hbm.at[pl.ds(off, _GK)], idx_v)
                pltpu.async_copy(table_hbm.at[idx_v], rows_v, sem).wait()
                pltpu.sync_copy(rows_v, out_hbm.at[pl.ds(off, _GK)])

            return carry

        lax.fori_loop(0, (_GCHUNKS + 31) // 32, body, 0)

    return gather_k


def _gather_rows(table, z):
    return _gather_rows_k()(table, z)


_EK = 80
_EPT = E // (NC * NS)
_ECHUNKS = _EPT // _EK
_ZR = 80
_ZCHUNKS = N // _ZR


@functools.cache
def _edge_agg_k():
    @functools.partial(
        pl.kernel,
        out_type=jax.ShapeDtypeStruct((NC, N, H), jnp.float32),
        mesh=_sc_mesh(),
        scratch_types=[
            pltpu.VMEM((_EK,), jnp.int32),
            pltpu.VMEM((_EK,), jnp.int32),
            pltpu.VMEM((_EK, H), jnp.float32),
            pltpu.VMEM((_ZR, H), jnp.float32),
            pltpu.VMEM_SHARED((N, H), jnp.float32),
            pltpu.SemaphoreType.DMA,
        ],
    )
    def agg_k(x_hbm, src_hbm, dst_hbm, out_hbm, src_v, dst_v, rows_v,
              stage_v, agg_sh, sem):
        cid = lax.axis_index("c")
        sid = lax.axis_index("s")

        def zbody(i, carry):
            for j in range(H // LANES):
                stage_v[i, pl.ds(LANES * j, LANES)] = jnp.zeros((LANES,),
                                                                jnp.float32)
            return carry

        lax.fori_loop(0, _ZR, zbody, 0)

        def zcopy(j, carry):
            c = sid + NS * j

            @pl.when(c < _ZCHUNKS)
            def _():
                pltpu.sync_copy(stage_v, agg_sh.at[pl.ds(c * _ZR, _ZR)])

            return carry

        lax.fori_loop(0, (_ZCHUNKS + NS - 1) // NS, zcopy, 0)
        plsc.subcore_barrier()

        ebase = cid * (E // NC) + sid * _EPT

        def body(j, carry):
            off = ebase + j * _EK
            pltpu.sync_copy(src_hbm.at[pl.ds(off, _EK)], src_v)
            pltpu.sync_copy(dst_hbm.at[pl.ds(off, _EK)], dst_v)
            pltpu.async_copy(x_hbm.at[src_v], rows_v, sem).wait()
            pltpu.sync_copy(rows_v, agg_sh.at[dst_v], add=True)
            return carry

        lax.fori_loop(0, _ECHUNKS, body, 0)
        plsc.subcore_barrier()

        def fcopy(j, carry):
            c = sid + NS * j

            @pl.when(c < _ZCHUNKS)
            def _():
                sl = pl.ds(c * _ZR, _ZR)
                pltpu.sync_copy(agg_sh.at[sl], stage_v)
                pltpu.sync_copy(stage_v, out_hbm.at[cid, sl])

            return carry

        lax.fori_loop(0, (_ZCHUNKS + NS - 1) // NS, fcopy, 0)

    return agg_k


def _edge_agg(x, src, dst):
    return _edge_agg_k()(x, src, dst)


_BR = 1000


def _mlp_body(x_ref, a_ref, w1_ref, b1_ref, w2_ref, b2_ref, g_ref, bt_ref,
              o_ref):
    h = x_ref[...] + a_ref[0] + a_ref[1]
    h = jnp.maximum(
        jnp.dot(h, w1_ref[...], preferred_element_type=jnp.float32)
        + b1_ref[...], 0.0)
    h = jnp.maximum(
        jnp.dot(h, w2_ref[...], preferred_element_type=jnp.float32)
        + b2_ref[...], 0.0)
    mu = jnp.mean(h, axis=1, keepdims=True)
    d = h - mu
    var = jnp.mean(d * d, axis=1, keepdims=True)
    o_ref[...] = d * lax.rsqrt(var + 1e-5) * g_ref[...] + bt_ref[...]


def _mlp(x, a, w1, b1, w2, b2, g, bt):
    return pl.pallas_call(
        _mlp_body,
        grid=(N // _BR,),
        in_specs=[
            pl.BlockSpec((_BR, H), lambda i: (i, 0)),
            pl.BlockSpec((NC, _BR, H), lambda i: (0, i, 0)),
            pl.BlockSpec((H, H), lambda i: (0, 0)),
            pl.BlockSpec((1, H), lambda i: (0, 0)),
            pl.BlockSpec((H, H), lambda i: (0, 0)),
            pl.BlockSpec((1, H), lambda i: (0, 0)),
            pl.BlockSpec((1, H), lambda i: (0, 0)),
            pl.BlockSpec((1, H), lambda i: (0, 0)),
        ],
        out_specs=pl.BlockSpec((_BR, H), lambda i: (i, 0)),
        out_shape=jax.ShapeDtypeStruct((N, H), jnp.float32),
    )(x, a, w1, b1.reshape(1, H), w2, b2.reshape(1, H), g.reshape(1, H),
      bt.reshape(1, H))


def _head_body(x1_ref, x2_ref, x3_ref, b_ref, l1w_ref, l1b_ref, l2w_ref,
               l2b_ref, o_ref, acc_ref, cnt_ref):
    i = pl.program_id(0)

    @pl.when(i == 0)
    def _():
        acc_ref[...] = jnp.zeros_like(acc_ref)
        cnt_ref[...] = jnp.zeros_like(cnt_ref)

    b = b_ref[0]
    oh = (lax.broadcasted_iota(jnp.int32, (NG, _BR), 0) == b).astype(
        jnp.float32)
    xc = jnp.concatenate([x1_ref[...], x2_ref[...], x3_ref[...]], axis=1)
    acc_ref[...] += jnp.dot(oh, xc, preferred_element_type=jnp.float32)
    cnt_ref[...] += jnp.sum(oh, axis=1, keepdims=True)

    @pl.when(i == N // _BR - 1)
    def _():
        pooled = acc_ref[...] / jnp.maximum(cnt_ref[...], 1.0)
        h = jnp.maximum(
            jnp.dot(pooled, l1w_ref[...], preferred_element_type=jnp.float32)
            + l1b_ref[...], 0.0)
        o_ref[...] = (jnp.sum(h * l2w_ref[...], axis=1, keepdims=True)
                      + l2b_ref[...])


def _head(x1, x2, x3, batch, l1w, l1b, l2w, l2b):
    return pl.pallas_call(
        _head_body,
        grid=(N // _BR,),
        in_specs=[
            pl.BlockSpec((_BR, H), lambda i: (i, 0)),
            pl.BlockSpec((_BR, H), lambda i: (i, 0)),
            pl.BlockSpec((_BR, H), lambda i: (i, 0)),
            pl.BlockSpec((1, 1, _BR), lambda i: (i, 0, 0)),
            pl.BlockSpec((3 * H, H), lambda i: (0, 0)),
            pl.BlockSpec((1, H), lambda i: (0, 0)),
            pl.BlockSpec((1, H), lambda i: (0, 0)),
            pl.BlockSpec((1, 1), lambda i: (0, 0)),
        ],
        out_specs=pl.BlockSpec((NG, 1), lambda i: (0, 0)),
        out_shape=jax.ShapeDtypeStruct((NG, 1), jnp.float32),
        scratch_shapes=[
            pltpu.VMEM((NG, 3 * H), jnp.float32),
            pltpu.VMEM((NG, 1), jnp.float32),
        ],
    )(x1, x2, x3, batch.reshape(N // _BR, 1, _BR), l1w, l1b.reshape(1, H),
      l2w.reshape(1, H), l2b.reshape(1, 1))


def kernel(z, edge_index, batch, z_table,
           w1_0, b1_0, w2_0, b2_0, g_0, bt_0,
           w1_1, b1_1, w2_1, b2_1, g_1, bt_1,
           w1_2, b1_2, w2_2, b2_2, g_2, bt_2,
           lin1_w, lin1_b, lin2_w, lin2_b):
    src = edge_index[0]
    dst = edge_index[1]
    x0 = _gather_rows(z_table, z)
    a0 = _edge_agg(x0, src, dst)
    x1 = _mlp(x0, a0, w1_0, b1_0, w2_0, b2_0, g_0, bt_0)
    a1 = _edge_agg(x1, src, dst)
    x2 = _mlp(x1, a1, w1_1, b1_1, w2_1, b2_1, g_1, bt_1)
    a2 = _edge_agg(x2, src, dst)
    x3 = _mlp(x2, a2, w1_2, b1_2, w2_2, b2_2, g_2, bt_2)
    return _head(x1, x2, x3, batch, lin1_w, lin1_b, lin2_w, lin2_b)


# speedup vs baseline: 12.6310x; 2.8333x over previous
import functools

import jax
import jax.numpy as jnp
from jax import lax
from jax.experimental import pallas as pl
from jax.experimental.pallas import tpu as pltpu
from jax.experimental.pallas import tpu_sc as plsc

N = 10000
E = 320000
H = 128
MAXZ = 1000
NG = 64

NC = 2
NS = 16
LANES = 16


@functools.cache
def _sc_mesh():
    return plsc.VectorSubcoreMesh(core_axis_name="c", subcore_axis_name="s",
                                  num_cores=NC, num_subcores=NS)


_GK = 80
_GCHUNKS = N // _GK


@functools.cache
def _gather_rows_k():
    @functools.partial(
        pl.kernel,
        out_type=jax.ShapeDtypeStruct((N, H), jnp.float32),
        mesh=_sc_mesh(),
        scratch_types=[
            pltpu.VMEM((_GK,), jnp.int32),
            pltpu.VMEM((_GK, H), jnp.float32),
            pltpu.SemaphoreType.DMA,
        ],
    )
    def gather_k(table_hbm, z_hbm, out_hbm, idx_v, rows_v, sem):
        cid = lax.axis_index("c")
        sid = lax.axis_index("s")
        wid = sid * NC + cid

        def body(j, carry):
            c = wid + 32 * j

            @pl.when(c < _GCHUNKS)
            def _():
                off = c * _GK
                pltpu.sync_copy(z_hbm.at[pl.ds(off, _GK)], idx_v)
                pltpu.async_copy(table_hbm.at[idx_v], rows_v, sem).wait()
                pltpu.sync_copy(rows_v, out_hbm.at[pl.ds(off, _GK)])

            return carry

        lax.fori_loop(0, (_GCHUNKS + 31) // 32, body, 0)

    return gather_k


def _gather_rows(table, z):
    return _gather_rows_k()(table, z)


_EK = 80
_EPT = E // (NC * NS)
_ECHUNKS = _EPT // _EK
_NB = 4
_NI = 2 * _NB
_ZR = 80
_ZCHUNKS = N // _ZR


@functools.cache
def _edge_agg_k():
    @functools.partial(
        pl.kernel,
        out_type=jax.ShapeDtypeStruct((NC, N, H), jnp.float32),
        mesh=_sc_mesh(),
        scratch_types=[
            [pltpu.VMEM((_EK,), jnp.int32)] * _NI,
            [pltpu.VMEM((_EK,), jnp.int32)] * _NI,
            [pltpu.VMEM((_EK, H), jnp.float32)] * _NB,
            pltpu.VMEM_SHARED((N, H), jnp.float32),
            [pltpu.SemaphoreType.DMA] * _NB,
            [pltpu.SemaphoreType.DMA] * _NI,
        ],
    )
    def agg_k(x_hbm, src_hbm, dst_hbm, out_hbm, sidx, didx, bufs,
              agg_sh, sems, isems):
        cid = lax.axis_index("c")
        sid = lax.axis_index("s")

        def zbody(i, carry):
            for j in range(H // LANES):
                bufs[0][i, pl.ds(LANES * j, LANES)] = jnp.zeros(
                    (LANES,), jnp.float32)
            return carry

        lax.fori_loop(0, _ZR, zbody, 0)

        def zcopy(j, carry):
            c = sid + NS * j

            @pl.when(c < _ZCHUNKS)
            def _():
                pltpu.sync_copy(bufs[0].at[pl.ds(0, _ZR)],
                                agg_sh.at[pl.ds(c * _ZR, _ZR)])

            return carry

        lax.fori_loop(0, (_ZCHUNKS + NS - 1) // NS, zcopy, 0)
        plsc.subcore_barrier()

        ebase = (cid * NS + sid) * _EPT

        def li(j, k):
            off = ebase + j * _EK
            pltpu.async_copy(src_hbm.at[pl.ds(off, _EK)], sidx[k],
                             isems[k])
            pltpu.async_copy(dst_hbm.at[pl.ds(off, _EK)], didx[k],
                             isems[k])

        def wi(k):
            pltpu.make_async_copy(src_hbm.at[pl.ds(ebase, _EK)], sidx[k],
                                  isems[k]).wait()
            pltpu.make_async_copy(dst_hbm.at[pl.ds(ebase, _EK)], didx[k],
                                  isems[k]).wait()

        def sg(k, b):
            pltpu.async_copy(x_hbm.at[sidx[k]], bufs[b], sems[b])

        def wg(b):
            pltpu.make_async_copy(x_hbm.at[sidx[0]], bufs[b],
                                  sems[b]).wait()

        for j in range(_NI):
            li(j, j)
        for j in range(_NB):
            wi(j)
            sg(j, j)

        def body(i, carry):
            for b in range(_NI):
                j = i * _NI + b
                bb = b % _NB
                wg(bb)
                pltpu.sync_copy(bufs[bb], agg_sh.at[didx[b]], add=True)

                @pl.when(j + _NI < _ECHUNKS)
                def _():
                    li(j + _NI, b)

                k2 = (b + _NB) % _NI
                wi(k2)
                sg(k2, bb)
            return carry

        lax.fori_loop(0, _ECHUNKS // _NI, body, 0)
        for j in range((_ECHUNKS // _NI) * _NI, _ECHUNKS):
            b = j % _NI
            bb = b % _NB
            wg(bb)
            pltpu.sync_copy(bufs[bb], agg_sh.at[didx[b]], add=True)
            if j + _NB < _ECHUNKS:
                k2 = (b + _NB) % _NI
                wi(k2)
                sg(k2, bb)
        plsc.subcore_barrier()

        def fcopy(j, carry):
            c = sid + NS * j

            @pl.when(c < _ZCHUNKS)
            def _():
                sl = pl.ds(c * _ZR, _ZR)
                pltpu.sync_copy(agg_sh.at[sl], bufs[0].at[pl.ds(0, _ZR)])
                pltpu.sync_copy(bufs[0].at[pl.ds(0, _ZR)],
                                out_hbm.at[cid, sl])

            return carry

        lax.fori_loop(0, (_ZCHUNKS + NS - 1) // NS, fcopy, 0)

    return agg_k


def _edge_agg(x, src, dst):
    return _edge_agg_k()(x, src, dst)


_BR = 1000


def _mlp_body(x_ref, a_ref, w1_ref, b1_ref, w2_ref, b2_ref, g_ref, bt_ref,
              o_ref):
    h = x_ref[...] + a_ref[0] + a_ref[1]
    h = jnp.maximum(
        jnp.dot(h, w1_ref[...], preferred_element_type=jnp.float32)
        + b1_ref[...], 0.0)
    h = jnp.maximum(
        jnp.dot(h, w2_ref[...], preferred_element_type=jnp.float32)
        + b2_ref[...], 0.0)
    mu = jnp.mean(h, axis=1, keepdims=True)
    d = h - mu
    var = jnp.mean(d * d, axis=1, keepdims=True)
    o_ref[...] = d * lax.rsqrt(var + 1e-5) * g_ref[...] + bt_ref[...]


def _mlp(x, a, w1, b1, w2, b2, g, bt):
    return pl.pallas_call(
        _mlp_body,
        grid=(N // _BR,),
        in_specs=[
            pl.BlockSpec((_BR, H), lambda i: (i, 0)),
            pl.BlockSpec((NC, _BR, H), lambda i: (0, i, 0)),
            pl.BlockSpec((H, H), lambda i: (0, 0)),
            pl.BlockSpec((1, H), lambda i: (0, 0)),
            pl.BlockSpec((H, H), lambda i: (0, 0)),
            pl.BlockSpec((1, H), lambda i: (0, 0)),
            pl.BlockSpec((1, H), lambda i: (0, 0)),
            pl.BlockSpec((1, H), lambda i: (0, 0)),
        ],
        out_specs=pl.BlockSpec((_BR, H), lambda i: (i, 0)),
        out_shape=jax.ShapeDtypeStruct((N, H), jnp.float32),
    )(x, a, w1, b1.reshape(1, H), w2, b2.reshape(1, H), g.reshape(1, H),
      bt.reshape(1, H))


def _head_body(x1_ref, x2_ref, x3_ref, b_ref, l1w_ref, l1b_ref, l2w_ref,
               l2b_ref, o_ref, acc_ref, cnt_ref):
    i = pl.program_id(0)

    @pl.when(i == 0)
    def _():
        acc_ref[...] = jnp.zeros_like(acc_ref)
        cnt_ref[...] = jnp.zeros_like(cnt_ref)

    b = b_ref[0]
    oh = (lax.broadcasted_iota(jnp.int32, (NG, _BR), 0) == b).astype(
        jnp.float32)
    xc = jnp.concatenate([x1_ref[...], x2_ref[...], x3_ref[...]], axis=1)
    acc_ref[...] += jnp.dot(oh, xc, preferred_element_type=jnp.float32)
    cnt_ref[...] += jnp.sum(oh, axis=1, keepdims=True)

    @pl.when(i == N // _BR - 1)
    def _():
        pooled = acc_ref[...] / jnp.maximum(cnt_ref[...], 1.0)
        h = jnp.maximum(
            jnp.dot(pooled, l1w_ref[...], preferred_element_type=jnp.float32)
            + l1b_ref[...], 0.0)
        o_ref[...] = (jnp.sum(h * l2w_ref[...], axis=1, keepdims=True)
                      + l2b_ref[...])


def _head(x1, x2, x3, batch, l1w, l1b, l2w, l2b):
    return pl.pallas_call(
        _head_body,
        grid=(N // _BR,),
        in_specs=[
            pl.BlockSpec((_BR, H), lambda i: (i, 0)),
            pl.BlockSpec((_BR, H), lambda i: (i, 0)),
            pl.BlockSpec((_BR, H), lambda i: (i, 0)),
            pl.BlockSpec((1, 1, _BR), lambda i: (i, 0, 0)),
            pl.BlockSpec((3 * H, H), lambda i: (0, 0)),
            pl.BlockSpec((1, H), lambda i: (0, 0)),
            pl.BlockSpec((1, H), lambda i: (0, 0)),
            pl.BlockSpec((1, 1), lambda i: (0, 0)),
        ],
        out_specs=pl.BlockSpec((NG, 1), lambda i: (0, 0)),
        out_shape=jax.ShapeDtypeStruct((NG, 1), jnp.float32),
        scratch_shapes=[
            pltpu.VMEM((NG, 3 * H), jnp.float32),
            pltpu.VMEM((NG, 1), jnp.float32),
        ],
    )(x1, x2, x3, batch.reshape(N // _BR, 1, _BR), l1w, l1b.reshape(1, H),
      l2w.reshape(1, H), l2b.reshape(1, 1))


def kernel(z, edge_index, batch, z_table,
           w1_0, b1_0, w2_0, b2_0, g_0, bt_0,
           w1_1, b1_1, w2_1, b2_1, g_1, bt_1,
           w1_2, b1_2, w2_2, b2_2, g_2, bt_2,
           lin1_w, lin1_b, lin2_w, lin2_b):
    src = edge_index[0]
    dst = edge_index[1]
    x0 = _gather_rows(z_table, z)
    a0 = _edge_agg(x0, src, dst)
    x1 = _mlp(x0, a0, w1_0, b1_0, w2_0, b2_0, g_0, bt_0)
    a1 = _edge_agg(x1, src, dst)
    x2 = _mlp(x1, a1, w1_1, b1_1, w2_1, b2_1, g_1, bt_1)
    a2 = _edge_agg(x2, src, dst)
    x3 = _mlp(x2, a2, w1_2, b1_2, w2_2, b2_2, g_2, bt_2)
    return _head(x1, x2, x3, batch, lin1_w, lin1_b, lin2_w, lin2_b)


# speedup vs baseline: 13.2536x; 1.0493x over previous
import functools

import jax
import jax.numpy as jnp
from jax import lax
from jax.experimental import pallas as pl
from jax.experimental.pallas import tpu as pltpu
from jax.experimental.pallas import tpu_sc as plsc

N = 10000
E = 320000
H = 128
MAXZ = 1000
NG = 64

NC = 2
NS = 16
LANES = 16


@functools.cache
def _sc_mesh():
    return plsc.VectorSubcoreMesh(core_axis_name="c", subcore_axis_name="s",
                                  num_cores=NC, num_subcores=NS)


_GK = 80
_GCHUNKS = N // _GK


@functools.cache
def _gather_rows_k():
    @functools.partial(
        pl.kernel,
        out_type=jax.ShapeDtypeStruct((N, H), jnp.float32),
        mesh=_sc_mesh(),
        scratch_types=[
            pltpu.VMEM((_GK,), jnp.int32),
            pltpu.VMEM((_GK, H), jnp.float32),
            pltpu.SemaphoreType.DMA,
        ],
    )
    def gather_k(table_hbm, z_hbm, out_hbm, idx_v, rows_v, sem):
        cid = lax.axis_index("c")
        sid = lax.axis_index("s")
        wid = sid * NC + cid

        def body(j, carry):
            c = wid + 32 * j

            @pl.when(c < _GCHUNKS)
            def _():
                off = c * _GK
                pltpu.sync_copy(z_hbm.at[pl.ds(off, _GK)], idx_v)
                pltpu.async_copy(table_hbm.at[idx_v], rows_v, sem).wait()
                pltpu.sync_copy(rows_v, out_hbm.at[pl.ds(off, _GK)])

            return carry

        lax.fori_loop(0, (_GCHUNKS + 31) // 32, body, 0)

    return gather_k


def _gather_rows(table, z):
    return _gather_rows_k()(table, z)


_EK = 80
_EPT = E // (NC * NS)
_ECHUNKS = _EPT // _EK
_NB = 4
_NI = 2 * _NB
_ZR = 80
_ZCHUNKS = N // _ZR


@functools.cache
def _edge_agg_k():
    @functools.partial(
        pl.kernel,
        out_type=jax.ShapeDtypeStruct((NC, N, H), jnp.float32),
        mesh=_sc_mesh(),
        scratch_types=[
            [pltpu.VMEM((_EK,), jnp.int32)] * _NI,
            [pltpu.VMEM((_EK,), jnp.int32)] * _NI,
            [pltpu.VMEM((_EK, H), jnp.float32)] * _NB,
            pltpu.VMEM_SHARED((N, H), jnp.float32),
            [pltpu.SemaphoreType.DMA] * _NB,
            [pltpu.SemaphoreType.DMA] * _NI,
        ],
    )
    def agg_k(x_hbm, src_hbm, dst_hbm, out_hbm, sidx, didx, bufs,
              agg_sh, sems, isems):
        cid = lax.axis_index("c")
        sid = lax.axis_index("s")
        ebase = (cid * NS + sid) * _EPT

        def li(j, k):
            off = ebase + j * _EK
            pltpu.async_copy(src_hbm.at[pl.ds(off, _EK)], sidx[k],
                             isems[k])
            pltpu.async_copy(dst_hbm.at[pl.ds(off, _EK)], didx[k],
                             isems[k])

        def wi(k):
            pltpu.make_async_copy(src_hbm.at[pl.ds(ebase, _EK)], sidx[k],
                                  isems[k]).wait()
            pltpu.make_async_copy(dst_hbm.at[pl.ds(ebase, _EK)], didx[k],
                                  isems[k]).wait()

        def sg(k, b):
            pltpu.async_copy(x_hbm.at[sidx[k]], bufs[b], sems[b])

        def wg(b):
            pltpu.make_async_copy(x_hbm.at[sidx[0]], bufs[b],
                                  sems[b]).wait()

        for j in range(_NI):
            li(j, j)
        for j in range(1, _NB):
            wi(j)
            sg(j, j)

        def zbody(i, carry):
            for j in range(H // LANES):
                bufs[0][i, pl.ds(LANES * j, LANES)] = jnp.zeros(
                    (LANES,), jnp.float32)
            return carry

        lax.fori_loop(0, _ZR, zbody, 0)

        def zcopy(j, carry):
            c = sid + NS * j

            @pl.when(c < _ZCHUNKS)
            def _():
                pltpu.sync_copy(bufs[0].at[pl.ds(0, _ZR)],
                                agg_sh.at[pl.ds(c * _ZR, _ZR)])

            return carry

        lax.fori_loop(0, (_ZCHUNKS + NS - 1) // NS, zcopy, 0)
        plsc.subcore_barrier()

        wi(0)
        sg(0, 0)

        def body(i, carry):
            for b in range(_NI):
                j = i * _NI + b
                bb = b % _NB
                wg(bb)
                pltpu.sync_copy(bufs[bb], agg_sh.at[didx[b]], add=True)

                @pl.when(j + _NI < _ECHUNKS)
                def _():
                    li(j + _NI, b)

                k2 = (b + _NB) % _NI
                wi(k2)
                sg(k2, bb)
            return carry

        lax.fori_loop(0, _ECHUNKS // _NI, body, 0)
        for j in range((_ECHUNKS // _NI) * _NI, _ECHUNKS):
            b = j % _NI
            bb = b % _NB
            wg(bb)
            pltpu.sync_copy(bufs[bb], agg_sh.at[didx[b]], add=True)
            if j + _NB < _ECHUNKS:
                k2 = (b + _NB) % _NI
                wi(k2)
                sg(k2, bb)
        plsc.subcore_barrier()

        def fcopy(j, carry):
            c = sid + NS * j

            @pl.when(c < _ZCHUNKS)
            def _():
                sl = pl.ds(c * _ZR, _ZR)
                pltpu.sync_copy(agg_sh.at[sl], out_hbm.at[cid, sl])

            return carry

        lax.fori_loop(0, (_ZCHUNKS + NS - 1) // NS, fcopy, 0)

    return agg_k


def _edge_agg(x, src, dst):
    return _edge_agg_k()(x, src, dst)


_BR = 1000


def _mlp_body(x_ref, a_ref, w1_ref, b1_ref, w2_ref, b2_ref, g_ref, bt_ref,
              o_ref):
    h = x_ref[...] + a_ref[0] + a_ref[1]
    h = jnp.maximum(
        jnp.dot(h, w1_ref[...], preferred_element_type=jnp.float32)
        + b1_ref[...], 0.0)
    h = jnp.maximum(
        jnp.dot(h, w2_ref[...], preferred_element_type=jnp.float32)
        + b2_ref[...], 0.0)
    mu = jnp.mean(h, axis=1, keepdims=True)
    d = h - mu
    var = jnp.mean(d * d, axis=1, keepdims=True)
    o_ref[...] = d * lax.rsqrt(var + 1e-5) * g_ref[...] + bt_ref[...]


def _mlp(x, a, w1, b1, w2, b2, g, bt):
    return pl.pallas_call(
        _mlp_body,
        grid=(N // _BR,),
        in_specs=[
            pl.BlockSpec((_BR, H), lambda i: (i, 0)),
            pl.BlockSpec((NC, _BR, H), lambda i: (0, i, 0)),
            pl.BlockSpec((H, H), lambda i: (0, 0)),
            pl.BlockSpec((1, H), lambda i: (0, 0)),
            pl.BlockSpec((H, H), lambda i: (0, 0)),
            pl.BlockSpec((1, H), lambda i: (0, 0)),
            pl.BlockSpec((1, H), lambda i: (0, 0)),
            pl.BlockSpec((1, H), lambda i: (0, 0)),
        ],
        out_specs=pl.BlockSpec((_BR, H), lambda i: (i, 0)),
        out_shape=jax.ShapeDtypeStruct((N, H), jnp.float32),
    )(x, a, w1, b1.reshape(1, H), w2, b2.reshape(1, H), g.reshape(1, H),
      bt.reshape(1, H))


def _mlp_head_body(x2_ref, a_ref, w1_ref, b1_ref, w2_ref, b2_ref, g_ref,
                   bt_ref, x1_ref, b_ref, l1w_ref, l1b_ref, l2w_ref,
                   l2b_ref, o_ref, acc_ref, cnt_ref):
    i = pl.program_id(0)

    @pl.when(i == 0)
    def _():
        acc_ref[...] = jnp.zeros_like(acc_ref)
        cnt_ref[...] = jnp.zeros_like(cnt_ref)

    h = x2_ref[...] + a_ref[0] + a_ref[1]
    h = jnp.maximum(
        jnp.dot(h, w1_ref[...], preferred_element_type=jnp.float32)
        + b1_ref[...], 0.0)
    h = jnp.maximum(
        jnp.dot(h, w2_ref[...], preferred_element_type=jnp.float32)
        + b2_ref[...], 0.0)
    mu = jnp.mean(h, axis=1, keepdims=True)
    d = h - mu
    var = jnp.mean(d * d, axis=1, keepdims=True)
    x3 = d * lax.rsqrt(var + 1e-5) * g_ref[...] + bt_ref[...]

    b = b_ref[0]
    oh = (lax.broadcasted_iota(jnp.int32, (NG, _BR), 0) == b).astype(
        jnp.float32)
    xc = jnp.concatenate([x1_ref[...], x2_ref[...], x3], axis=1)
    acc_ref[...] += jnp.dot(oh, xc, preferred_element_type=jnp.float32)
    cnt_ref[...] += jnp.sum(oh, axis=1, keepdims=True)

    @pl.when(i == N // _BR - 1)
    def _():
        pooled = acc_ref[...] / jnp.maximum(cnt_ref[...], 1.0)
        hh = jnp.maximum(
            jnp.dot(pooled, l1w_ref[...], preferred_element_type=jnp.float32)
            + l1b_ref[...], 0.0)
        o_ref[...] = (jnp.sum(hh * l2w_ref[...], axis=1, keepdims=True)
                      + l2b_ref[...])


def _mlp_head(x2, a, w1, b1, w2, b2, g, bt, x1, batch, l1w, l1b, l2w, l2b):
    return pl.pallas_call(
        _mlp_head_body,
        grid=(N // _BR,),
        in_specs=[
            pl.BlockSpec((_BR, H), lambda i: (i, 0)),
            pl.BlockSpec((NC, _BR, H), lambda i: (0, i, 0)),
            pl.BlockSpec((H, H), lambda i: (0, 0)),
            pl.BlockSpec((1, H), lambda i: (0, 0)),
            pl.BlockSpec((H, H), lambda i: (0, 0)),
            pl.BlockSpec((1, H), lambda i: (0, 0)),
            pl.BlockSpec((1, H), lambda i: (0, 0)),
            pl.BlockSpec((1, H), lambda i: (0, 0)),
            pl.BlockSpec((_BR, H), lambda i: (i, 0)),
            pl.BlockSpec((1, 1, _BR), lambda i: (i, 0, 0)),
            pl.BlockSpec((3 * H, H), lambda i: (0, 0)),
            pl.BlockSpec((1, H), lambda i: (0, 0)),
            pl.BlockSpec((1, H), lambda i: (0, 0)),
            pl.BlockSpec((1, 1), lambda i: (0, 0)),
        ],
        out_specs=pl.BlockSpec((NG, 1), lambda i: (0, 0)),
        out_shape=jax.ShapeDtypeStruct((NG, 1), jnp.float32),
        scratch_shapes=[
            pltpu.VMEM((NG, 3 * H), jnp.float32),
            pltpu.VMEM((NG, 1), jnp.float32),
        ],
    )(x2, a, w1, b1.reshape(1, H), w2, b2.reshape(1, H), g.reshape(1, H),
      bt.reshape(1, H), x1, batch.reshape(N // _BR, 1, _BR), l1w,
      l1b.reshape(1, H), l2w.reshape(1, H), l2b.reshape(1, 1))


def kernel(z, edge_index, batch, z_table,
           w1_0, b1_0, w2_0, b2_0, g_0, bt_0,
           w1_1, b1_1, w2_1, b2_1, g_1, bt_1,
           w1_2, b1_2, w2_2, b2_2, g_2, bt_2,
           lin1_w, lin1_b, lin2_w, lin2_b):
    src = edge_index[0]
    dst = edge_index[1]
    x0 = _gather_rows(z_table, z)
    a0 = _edge_agg(x0, src, dst)
    x1 = _mlp(x0, a0, w1_0, b1_0, w2_0, b2_0, g_0, bt_0)
    a1 = _edge_agg(x1, src, dst)
    x2 = _mlp(x1, a1, w1_1, b1_1, w2_1, b2_1, g_1, bt_1)
    a2 = _edge_agg(x2, src, dst)
    return _mlp_head(x2, a2, w1_2, b1_2, w2_2, b2_2, g_2, bt_2,
                     x1, batch, lin1_w, lin1_b, lin2_w, lin2_b)


# speedup vs baseline: 13.5704x; 1.0239x over previous
import functools

import jax
import jax.numpy as jnp
from jax import lax
from jax.experimental import pallas as pl
from jax.experimental.pallas import tpu as pltpu
from jax.experimental.pallas import tpu_sc as plsc

N = 10000
E = 320000
H = 128
MAXZ = 1000
NG = 64

NC = 2
NS = 16
LANES = 16


@functools.cache
def _sc_mesh():
    return plsc.VectorSubcoreMesh(core_axis_name="c", subcore_axis_name="s",
                                  num_cores=NC, num_subcores=NS)


_GK = 80
_GCHUNKS = N // _GK


@functools.cache
def _gather_rows_k():
    @functools.partial(
        pl.kernel,
        out_type=jax.ShapeDtypeStruct((N, H), jnp.float32),
        mesh=_sc_mesh(),
        scratch_types=[
            pltpu.VMEM((_GK,), jnp.int32),
            pltpu.VMEM((_GK, H), jnp.float32),
            pltpu.SemaphoreType.DMA,
        ],
    )
    def gather_k(table_hbm, z_hbm, out_hbm, idx_v, rows_v, sem):
        cid = lax.axis_index("c")
        sid = lax.axis_index("s")
        wid = sid * NC + cid

        def body(j, carry):
            c = wid + 32 * j

            @pl.when(c < _GCHUNKS)
            def _():
                off = c * _GK
                pltpu.sync_copy(z_hbm.at[pl.ds(off, _GK)], idx_v)
                pltpu.async_copy(table_hbm.at[idx_v], rows_v, sem).wait()
                pltpu.sync_copy(rows_v, out_hbm.at[pl.ds(off, _GK)])

            return carry

        lax.fori_loop(0, (_GCHUNKS + 31) // 32, body, 0)

    return gather_k


def _gather_rows(table, z):
    return _gather_rows_k()(table, z)


_EK = 80
_EPT = E // (NC * NS)
_ECHUNKS = _EPT // _EK
_NB = 4
_NI = 2 * _NB
_ZR = 80
_ZCHUNKS = N // _ZR


@functools.cache
def _edge_agg_k():
    @functools.partial(
        pl.kernel,
        out_type=jax.ShapeDtypeStruct((NC, N, H), jnp.float32),
        mesh=_sc_mesh(),
        scratch_types=[
            [pltpu.VMEM((_EK,), jnp.int32)] * _NI,
            [pltpu.VMEM((_EK,), jnp.int32)] * _NI,
            [pltpu.VMEM((_EK, H), jnp.float32)] * _NB,
            pltpu.VMEM_SHARED((N, H), jnp.float32),
            [pltpu.SemaphoreType.DMA] * _NB,
            [pltpu.SemaphoreType.DMA] * _NI,
        ],
    )
    def agg_k(x_hbm, src_hbm, dst_hbm, out_hbm, sidx, didx, bufs,
              agg_sh, sems, isems):
        cid = lax.axis_index("c")
        sid = lax.axis_index("s")
        ebase = (cid * NS + sid) * _EPT

        def li(j, k):
            off = ebase + j * _EK
            pltpu.async_copy(src_hbm.at[pl.ds(off, _EK)], sidx[k],
                             isems[k])
            pltpu.async_copy(dst_hbm.at[pl.ds(off, _EK)], didx[k],
                             isems[k])

        def wi(k):
            pltpu.make_async_copy(src_hbm.at[pl.ds(ebase, _EK)], sidx[k],
                                  isems[k]).wait()
            pltpu.make_async_copy(dst_hbm.at[pl.ds(ebase, _EK)], didx[k],
                                  isems[k]).wait()

        def sg(k, b):
            pltpu.async_copy(x_hbm.at[sidx[k]], bufs[b], sems[b])

        def wg(b):
            pltpu.make_async_copy(x_hbm.at[sidx[0]], bufs[b],
                                  sems[b]).wait()

        for j in range(_NI):
            li(j, j)
        for j in range(1, _NB):
            wi(j)
            sg(j, j)

        def zbody(i, carry):
            for j in range(H // LANES):
                bufs[0][i, pl.ds(LANES * j, LANES)] = jnp.zeros(
                    (LANES,), jnp.float32)
            return carry

        lax.fori_loop(0, _ZR, zbody, 0)

        def zcopy(j, carry):
            c = sid + NS * j

            @pl.when(c < _ZCHUNKS)
            def _():
                pltpu.sync_copy(bufs[0].at[pl.ds(0, _ZR)],
                                agg_sh.at[pl.ds(c * _ZR, _ZR)])

            return carry

        lax.fori_loop(0, (_ZCHUNKS + NS - 1) // NS, zcopy, 0)
        plsc.subcore_barrier()

        wi(0)
        sg(0, 0)

        def body(i, carry):
            for b in range(_NI):
                j = i * _NI + b
                bb = b % _NB
                wg(bb)
                pltpu.sync_copy(bufs[bb], agg_sh.at[didx[b]], add=True)

                @pl.when(j + _NI < _ECHUNKS)
                def _():
                    li(j + _NI, b)

                k2 = (b + _NB) % _NI
                wi(k2)
                sg(k2, bb)
            return carry

        lax.fori_loop(0, _ECHUNKS // _NI, body, 0)
        for j in range((_ECHUNKS // _NI) * _NI, _ECHUNKS):
            b = j % _NI
            bb = b % _NB
            wg(bb)
            pltpu.sync_copy(bufs[bb], agg_sh.at[didx[b]], add=True)
            if j + _NB < _ECHUNKS:
                k2 = (b + _NB) % _NI
                wi(k2)
                sg(k2, bb)
        plsc.subcore_barrier()

        def fcopy(j, carry):
            c = sid + NS * j

            @pl.when(c < _ZCHUNKS)
            def _():
                sl = pl.ds(c * _ZR, _ZR)
                pltpu.sync_copy(agg_sh.at[sl], out_hbm.at[cid, sl])

            return carry

        lax.fori_loop(0, (_ZCHUNKS + NS - 1) // NS, fcopy, 0)

    return agg_k


def _edge_agg(x, src, dst):
    return _edge_agg_k()(x, src, dst)


_BR = 2000


def _mlp_body(x_ref, a_ref, w1_ref, b1_ref, w2_ref, b2_ref, g_ref, bt_ref,
              o_ref):
    h = x_ref[...] + a_ref[0] + a_ref[1]
    h = jnp.maximum(
        jnp.dot(h, w1_ref[...], preferred_element_type=jnp.float32)
        + b1_ref[...], 0.0)
    h = jnp.maximum(
        jnp.dot(h, w2_ref[...], preferred_element_type=jnp.float32)
        + b2_ref[...], 0.0)
    mu = jnp.mean(h, axis=1, keepdims=True)
    d = h - mu
    var = jnp.mean(d * d, axis=1, keepdims=True)
    o_ref[...] = d * lax.rsqrt(var + 1e-5) * g_ref[...] + bt_ref[...]


def _mlp(x, a, w1, b1, w2, b2, g, bt):
    return pl.pallas_call(
        _mlp_body,
        grid=(N // _BR,),
        in_specs=[
            pl.BlockSpec((_BR, H), lambda i: (i, 0)),
            pl.BlockSpec((NC, _BR, H), lambda i: (0, i, 0)),
            pl.BlockSpec((H, H), lambda i: (0, 0)),
            pl.BlockSpec((1, H), lambda i: (0, 0)),
            pl.BlockSpec((H, H), lambda i: (0, 0)),
            pl.BlockSpec((1, H), lambda i: (0, 0)),
            pl.BlockSpec((1, H), lambda i: (0, 0)),
            pl.BlockSpec((1, H), lambda i: (0, 0)),
        ],
        out_specs=pl.BlockSpec((_BR, H), lambda i: (i, 0)),
        out_shape=jax.ShapeDtypeStruct((N, H), jnp.float32),
    )(x, a, w1, b1.reshape(1, H), w2, b2.reshape(1, H), g.reshape(1, H),
      bt.reshape(1, H))


def _mlp_head_body(x2_ref, a_ref, w1_ref, b1_ref, w2_ref, b2_ref, g_ref,
                   bt_ref, x1_ref, b_ref, l1w_ref, l1b_ref, l2w_ref,
                   l2b_ref, o_ref, acc_ref, cnt_ref):
    i = pl.program_id(0)

    @pl.when(i == 0)
    def _():
        acc_ref[...] = jnp.zeros_like(acc_ref)
        cnt_ref[...] = jnp.zeros_like(cnt_ref)

    h = x2_ref[...] + a_ref[0] + a_ref[1]
    h = jnp.maximum(
        jnp.dot(h, w1_ref[...], preferred_element_type=jnp.float32)
        + b1_ref[...], 0.0)
    h = jnp.maximum(
        jnp.dot(h, w2_ref[...], preferred_element_type=jnp.float32)
        + b2_ref[...], 0.0)
    mu = jnp.mean(h, axis=1, keepdims=True)
    d = h - mu
    var = jnp.mean(d * d, axis=1, keepdims=True)
    x3 = d * lax.rsqrt(var + 1e-5) * g_ref[...] + bt_ref[...]

    b = b_ref[0]
    oh = (lax.broadcasted_iota(jnp.int32, (NG, _BR), 0) == b).astype(
        jnp.float32)
    xc = jnp.concatenate([x1_ref[...], x2_ref[...], x3], axis=1)
    acc_ref[...] += jnp.dot(oh, xc, preferred_element_type=jnp.float32)
    cnt_ref[...] += jnp.sum(oh, axis=1, keepdims=True)

    @pl.when(i == N // _BR - 1)
    def _():
        pooled = acc_ref[...] / jnp.maximum(cnt_ref[...], 1.0)
        hh = jnp.maximum(
            jnp.dot(pooled, l1w_ref[...], preferred_element_type=jnp.float32)
            + l1b_ref[...], 0.0)
        o_ref[...] = (jnp.sum(hh * l2w_ref[...], axis=1, keepdims=True)
                      + l2b_ref[...])


def _mlp_head(x2, a, w1, b1, w2, b2, g, bt, x1, batch, l1w, l1b, l2w, l2b):
    return pl.pallas_call(
        _mlp_head_body,
        grid=(N // _BR,),
        in_specs=[
            pl.BlockSpec((_BR, H), lambda i: (i, 0)),
            pl.BlockSpec((NC, _BR, H), lambda i: (0, i, 0)),
            pl.BlockSpec((H, H), lambda i: (0, 0)),
            pl.BlockSpec((1, H), lambda i: (0, 0)),
            pl.BlockSpec((H, H), lambda i: (0, 0)),
            pl.BlockSpec((1, H), lambda i: (0, 0)),
            pl.BlockSpec((1, H), lambda i: (0, 0)),
            pl.BlockSpec((1, H), lambda i: (0, 0)),
            pl.BlockSpec((_BR, H), lambda i: (i, 0)),
            pl.BlockSpec((1, 1, _BR), lambda i: (i, 0, 0)),
            pl.BlockSpec((3 * H, H), lambda i: (0, 0)),
            pl.BlockSpec((1, H), lambda i: (0, 0)),
            pl.BlockSpec((1, H), lambda i: (0, 0)),
            pl.BlockSpec((1, 1), lambda i: (0, 0)),
        ],
        out_specs=pl.BlockSpec((NG, 1), lambda i: (0, 0)),
        out_shape=jax.ShapeDtypeStruct((NG, 1), jnp.float32),
        scratch_shapes=[
            pltpu.VMEM((NG, 3 * H), jnp.float32),
            pltpu.VMEM((NG, 1), jnp.float32),
        ],
    )(x2, a, w1, b1.reshape(1, H), w2, b2.reshape(1, H), g.reshape(1, H),
      bt.reshape(1, H), x1, batch.reshape(N // _BR, 1, _BR), l1w,
      l1b.reshape(1, H), l2w.reshape(1, H), l2b.reshape(1, 1))


def kernel(z, edge_index, batch, z_table,
           w1_0, b1_0, w2_0, b2_0, g_0, bt_0,
           w1_1, b1_1, w2_1, b2_1, g_1, bt_1,
           w1_2, b1_2, w2_2, b2_2, g_2, bt_2,
           lin1_w, lin1_b, lin2_w, lin2_b):
    src = edge_index[0]
    dst = edge_index[1]
    x0 = _gather_rows(z_table, z)
    a0 = _edge_agg(x0, src, dst)
    x1 = _mlp(x0, a0, w1_0, b1_0, w2_0, b2_0, g_0, bt_0)
    a1 = _edge_agg(x1, src, dst)
    x2 = _mlp(x1, a1, w1_1, b1_1, w2_1, b2_1, g_1, bt_1)
    a2 = _edge_agg(x2, src, dst)
    return _mlp_head(x2, a2, w1_2, b1_2, w2_2, b2_2, g_2, bt_2,
                     x1, batch, lin1_w, lin1_b, lin2_w, lin2_b)
